# baseline (speedup 1.0000x reference)
import functools

import jax
import jax.numpy as jnp
from jax import lax
from jax.experimental import pallas as pl
from jax.experimental.pallas import tpu as pltpu
from jax.experimental.pallas import tpu_sc as plsc

N = 50000
E = 800000
D_IN = 128
D_H = 64
D_OUT = 128

NC = 2
NS = 16
LANES = 128
CHUNKS = 6336
EPAD = CHUNKS * LANES
NPAD = 51200
ROWS_PER_TILE = NPAD // NS

AGG_CHUNKS_PER_TILE = CHUNKS // NS
AGG_K = 2
AGG_PAIRS = AGG_CHUNKS_PER_TILE // (2 * AGG_K)

DEG_CHUNKS_PER_TILE = CHUNKS // (NC * NS)
DEG_K = 3
DEG_GROUPS = DEG_CHUNKS_PER_TILE // DEG_K

_mesh = plsc.VectorSubcoreMesh(
    core_axis_name="c", subcore_axis_name="s", num_cores=NC, num_subcores=NS)


@functools.partial(
    pl.kernel,
    out_type=(jax.ShapeDtypeStruct((NPAD,), jnp.float32),
              jax.ShapeDtypeStruct((NPAD,), jnp.float32)),
    mesh=_mesh,
    scratch_types=[
        pltpu.VMEM((DEG_K, LANES), jnp.int32),
        pltpu.VMEM((LANES,), jnp.float32),
        pltpu.VMEM((ROWS_PER_TILE,), jnp.float32),
        pltpu.VMEM_SHARED((NPAD,), jnp.float32),
        pltpu.SemaphoreType.DMA,
    ],
    compiler_params=pltpu.CompilerParams(use_tc_tiling_on_sc=False),
)
def _sc_degree(dst_hbm, out0_hbm, out1_hbm, didx, ones, zbuf, acc, ssem):
  c = lax.axis_index("c")
  s = lax.axis_index("s")
  one16 = jnp.ones((16,), jnp.float32)
  z16 = jnp.zeros((16,), jnp.float32)

  def zb_body(i, carry):
    zbuf[pl.ds(i * 16, 16)] = z16
    return carry

  lax.fori_loop(0, ROWS_PER_TILE // 16, zb_body, 0)
  for i in range(LANES // 16):
    ones[pl.ds(i * 16, 16)] = one16

  pltpu.sync_copy(zbuf, acc.at[pl.ds(s * ROWS_PER_TILE, ROWS_PER_TILE)])
  plsc.subcore_barrier()

  base = (c * NS + s) * DEG_CHUNKS_PER_TILE

  def group(g, carry):
    gb = base + g * DEG_K
    pltpu.sync_copy(dst_hbm.at[pl.ds(gb, DEG_K)], didx)
    handles = []
    for b in range(DEG_K):
      handles.append(
          pltpu.async_copy(ones, acc.at[didx.at[b]], ssem, add=True))
    for h in handles:
      h.wait()
    return carry

  lax.fori_loop(0, DEG_GROUPS, group, 0)
  plsc.subcore_barrier()
  sl = pl.ds(s * ROWS_PER_TILE, ROWS_PER_TILE)

  @pl.when(c == 0)
  def _():
    pltpu.sync_copy(acc.at[sl], out0_hbm.at[sl])

  @pl.when(c == 1)
  def _():
    pltpu.sync_copy(acc.at[sl], out1_hbm.at[sl])


@functools.partial(
    pl.kernel,
    out_type=jax.ShapeDtypeStruct((NC, NPAD, 32), jnp.float32),
    mesh=_mesh,
    scratch_types=[
        pltpu.VMEM((AGG_K, LANES), jnp.int32),
        pltpu.VMEM((AGG_K, LANES), jnp.int32),
        pltpu.VMEM((AGG_K, LANES), jnp.int32),
        pltpu.VMEM((AGG_K, LANES), jnp.int32),
        pltpu.VMEM((AGG_K, LANES, 32), jnp.bfloat16),
        pltpu.VMEM((AGG_K, LANES, 32), jnp.bfloat16),
        pltpu.VMEM((AGG_K, LANES, 32), jnp.float32),
        pltpu.VMEM((AGG_K, LANES, 32), jnp.float32),
        pltpu.VMEM_SHARED((NPAD, 32), jnp.float32),
        pltpu.SemaphoreType.DMA,
        pltpu.SemaphoreType.DMA,
        pltpu.SemaphoreType.DMA,
        pltpu.SemaphoreType.DMA,
    ],
    compiler_params=pltpu.CompilerParams(
        use_tc_tiling_on_sc=False, needs_layout_passes=False),
)
def _sc_agg(table_hbm, src_hbm, dst_hbm, out_hbm,
            sidxA, didxA, sidxB, didxB, rbfA, rbfB, rfA, rfB, acc,
            gsemA, gsemB, ssemA, ssemB):
  c = lax.axis_index("c")
  s = lax.axis_index("s")

  z16 = jnp.zeros((16,), jnp.float32)

  def zrow(i, carry):
    rfA[0, i, 0:16] = z16
    rfA[0, i, 16:32] = z16
    return carry

  lax.fori_loop(0, LANES, zrow, 0)

  def zcopy(k, carry):
    pltpu.sync_copy(
        rfA.at[0], acc.at[pl.ds(s * ROWS_PER_TILE + k * LANES, LANES)])
    return carry

  lax.fori_loop(0, ROWS_PER_TILE // LANES, zcopy, 0)
  plsc.subcore_barrier()

  base = s * AGG_CHUNKS_PER_TILE
  ev_idx = lax.iota(jnp.int32, 16) * 2
  od_idx = ev_idx + 1

  def load_and_gather(g, sidx, didx, rows, gsem):
    pltpu.sync_copy(src_hbm.at[c, pl.ds(g, AGG_K)], sidx)
    pltpu.sync_copy(dst_hbm.at[pl.ds(g, AGG_K)], didx)
    for b in range(AGG_K):
      pltpu.async_copy(table_hbm.at[sidx.at[b]], rows.at[b], gsem)

  def drain_gathers(sidx, rows, gsem):
    for b in range(AGG_K):
      pltpu.make_async_copy(table_hbm.at[sidx.at[b]], rows.at[b], gsem).wait()

  def convert(rbf, rf):
    for b in range(AGG_K):
      def conv_row(i, carry):
        v = rbf[b, i, 0:32]
        lo, hi = plsc.unpack(v, format=plsc.PackFormat.INTERLEAVED)
        plsc.store_scatter(rf.at[b, i], [ev_idx], lo)
        plsc.store_scatter(rf.at[b, i], [od_idx], hi)
        return carry

      lax.fori_loop(0, LANES, conv_row, 0)

  def fire_scatters(didx, rf, ssem):
    for b in range(AGG_K):
      pltpu.async_copy(rf.at[b], acc.at[didx.at[b]], ssem, add=True)

  def drain_scatters(didx, rf, ssem):
    for b in range(AGG_K):
      pltpu.make_async_copy(rf.at[b], acc.at[didx.at[b]], ssem).wait()

  load_and_gather(base, sidxA, didxA, rbfA, gsemA)

  def pair(i, carry):
    ga = base + 2 * AGG_K * i
    drain_gathers(sidxA, rbfA, gsemA)
    load_and_gather(ga + AGG_K, sidxB, didxB, rbfB, gsemB)

    @pl.when(i > 0)
    def _():
      drain_scatters(didxA, rfA, ssemA)

    convert(rbfA, rfA)
    fire_scatters(didxA, rfA, ssemA)

    drain_gathers(sidxB, rbfB, gsemB)

    @pl.when(i < AGG_PAIRS - 1)
    def _():
      load_and_gather(ga + 2 * AGG_K, sidxA, didxA, rbfA, gsemA)

    @pl.when(i > 0)
    def _():
      drain_scatters(didxB, rfB, ssemB)

    convert(rbfB, rfB)
    fire_scatters(didxB, rfB, ssemB)
    return carry

  lax.fori_loop(0, AGG_PAIRS, pair, 0)
  drain_scatters(didxA, rfA, ssemA)
  drain_scatters(didxB, rfB, ssemB)
  plsc.subcore_barrier()
  pltpu.sync_copy(acc.at[pl.ds(s * ROWS_PER_TILE, ROWS_PER_TILE)],
                  out_hbm.at[c, pl.ds(s * ROWS_PER_TILE, ROWS_PER_TILE)])


_R = 2000


def _b1_body(x_ref, w_ref, d0_ref, d1_ref, g_ref, dinv_ref):
  deg = d0_ref[...] + d1_ref[...] + 1.0
  dinv = lax.rsqrt(deg)
  h = jnp.dot(x_ref[...], w_ref[...], preferred_element_type=jnp.float32)
  g = h * dinv
  g_ref[0] = g[:, :32]
  g_ref[1] = g[:, 32:]
  dinv_ref[...] = dinv


def _tc_b1(x, w_in, d0, d1):
  return pl.pallas_call(
      _b1_body,
      grid=(N // _R,),
      in_specs=[
          pl.BlockSpec((_R, D_IN), lambda i: (i, 0)),
          pl.BlockSpec((D_IN, D_H), lambda i: (0, 0)),
          pl.BlockSpec((_R, 1), lambda i: (i, 0)),
          pl.BlockSpec((_R, 1), lambda i: (i, 0)),
      ],
      out_specs=[
          pl.BlockSpec((NC, _R, 32), lambda i: (0, i, 0)),
          pl.BlockSpec((_R, 1), lambda i: (i, 0)),
      ],
      out_shape=[
          jax.ShapeDtypeStruct((NC, N, 32), jnp.float32),
          jax.ShapeDtypeStruct((N, 1), jnp.float32),
      ],
  )(x, w_in, d0, d1)


def _b2_body(agg_ref, g_ref, dinv_ref, b_ref, w_ref, o_ref):
  a = jnp.concatenate([agg_ref[0], agg_ref[1]], axis=1)
  g = jnp.concatenate([g_ref[0], g_ref[1]], axis=1)
  dinv = dinv_ref[...]
  h = jax.nn.relu(dinv * (a + g) + b_ref[...])
  g2 = jnp.dot(h, w_ref[...], preferred_element_type=jnp.float32)
  g2 = g2 * dinv
  o_ref[0] = g2[:, :32]
  o_ref[1] = g2[:, 32:]


def _tc_b2(agg, g, dinv, b, w):
  return pl.pallas_call(
      _b2_body,
      grid=(N // _R,),
      in_specs=[
          pl.BlockSpec((NC, _R, 32), lambda i: (0, i, 0)),
          pl.BlockSpec((NC, _R, 32), lambda i: (0, i, 0)),
          pl.BlockSpec((_R, 1), lambda i: (i, 0)),
          pl.BlockSpec((1, D_H), lambda i: (0, 0)),
          pl.BlockSpec((D_H, D_H), lambda i: (0, 0)),
      ],
      out_specs=pl.BlockSpec((NC, _R, 32), lambda i: (0, i, 0)),
      out_shape=jax.ShapeDtypeStruct((NC, N, 32), jnp.float32),
  )(agg, g, dinv, b, w)


def _b3_body(agg_ref, g_ref, dinv_ref, b_ref, o_ref):
  dinv = dinv_ref[...]
  for k in range(NC):
    h = jax.nn.relu(
        dinv * (agg_ref[k] + g_ref[k]) + b_ref[0, k * 32:(k + 1) * 32])
    o_ref[k] = h * dinv


def _tc_b3(agg, g, dinv, b):
  return pl.pallas_call(
      _b3_body,
      grid=(N // _R,),
      in_specs=[
          pl.BlockSpec((NC, _R, 32), lambda i: (0, i, 0)),
          pl.BlockSpec((NC, _R, 32), lambda i: (0, i, 0)),
          pl.BlockSpec((_R, 1), lambda i: (i, 0)),
          pl.BlockSpec((1, D_H), lambda i: (0, 0)),
      ],
      out_specs=pl.BlockSpec((NC, _R, 32), lambda i: (0, i, 0)),
      out_shape=jax.ShapeDtypeStruct((NC, N, 32), jnp.float32),
  )(agg, g, dinv, b)


def _b4_body(agg_ref, g_ref, dinv_ref, w_ref, b_ref, o_ref):
  a = jnp.concatenate([agg_ref[0], agg_ref[1]], axis=1)
  g = jnp.concatenate([g_ref[0], g_ref[1]], axis=1)
  dinv = dinv_ref[...]
  s3 = dinv * (a + g)
  o_ref[...] = jnp.dot(
      s3, w_ref[...], preferred_element_type=jnp.float32) + b_ref[...]


def _tc_b4(agg, g, dinv, w, b):
  return pl.pallas_call(
      _b4_body,
      grid=(N // _R,),
      in_specs=[
          pl.BlockSpec((NC, _R, 32), lambda i: (0, i, 0)),
          pl.BlockSpec((NC, _R, 32), lambda i: (0, i, 0)),
          pl.BlockSpec((_R, 1), lambda i: (i, 0)),
          pl.BlockSpec((D_H, D_OUT), lambda i: (0, 0)),
          pl.BlockSpec((1, D_OUT), lambda i: (0, 0)),
      ],
      out_specs=pl.BlockSpec((_R, D_OUT), lambda i: (i, 0)),
      out_shape=jax.ShapeDtypeStruct((N, D_OUT), jnp.float32),
  )(agg, g, dinv, w, b)


def kernel(x, edge_index, W_in, b_in, W_h, b_h, W_out, b_out):
  src = edge_index[0]
  dst = edge_index[1]
  pad = EPAD - E
  src_p = jnp.concatenate([src, jnp.zeros((pad,), jnp.int32)])
  dst_p = jnp.concatenate([dst, jnp.full((pad,), N, jnp.int32)])
  srcs = jnp.stack([src_p, src_p + N]).reshape(NC, CHUNKS, LANES)
  dst3 = dst_p.reshape(CHUNKS, LANES)

  d0, d1 = _sc_degree(dst3)
  g1, dinv = _tc_b1(x, W_in, d0.reshape(NPAD, 1), d1.reshape(NPAD, 1))

  agg1 = _sc_agg(g1.reshape(NC * N, 32).astype(jnp.bfloat16), srcs, dst3)
  g2 = _tc_b2(agg1, g1, dinv, b_in.reshape(1, D_H), W_h)

  agg2 = _sc_agg(g2.reshape(NC * N, 32).astype(jnp.bfloat16), srcs, dst3)
  g3 = _tc_b3(agg2, g2, dinv, b_h.reshape(1, D_H))

  agg3 = _sc_agg(g3.reshape(NC * N, 32).astype(jnp.bfloat16), srcs, dst3)
  out = _tc_b4(agg3, g3, dinv, W_out, b_out.reshape(1, D_OUT))
  return out


# speedup vs baseline: 1.2976x; 1.2976x over previous
import functools

import jax
import jax.numpy as jnp
from jax import lax
from jax.experimental import pallas as pl
from jax.experimental.pallas import tpu as pltpu
from jax.experimental.pallas import tpu_sc as plsc

N = 50000
E = 800000
D_IN = 128
D_H = 64
D_OUT = 128

NC = 2
NS = 16
LANES = 128
CHUNKS = 6336
EPAD = CHUNKS * LANES
NPAD = 51200
ROWS_PER_TILE = NPAD // NS

AGG_CHUNKS_PER_TILE = CHUNKS // NS
AGG_BATCH = 36
AGG_BATCHES = AGG_CHUNKS_PER_TILE // AGG_BATCH
AGG_K = 4

DEG_CHUNKS_PER_TILE = CHUNKS // (NC * NS)
DEG_K = 3
DEG_GROUPS = DEG_CHUNKS_PER_TILE // DEG_K

_mesh = plsc.VectorSubcoreMesh(
    core_axis_name="c", subcore_axis_name="s", num_cores=NC, num_subcores=NS)


@functools.partial(
    pl.kernel,
    out_type=(jax.ShapeDtypeStruct((NPAD,), jnp.float32),
              jax.ShapeDtypeStruct((NPAD,), jnp.float32)),
    mesh=_mesh,
    scratch_types=[
        pltpu.VMEM((DEG_K, LANES), jnp.int32),
        pltpu.VMEM((LANES,), jnp.float32),
        pltpu.VMEM((ROWS_PER_TILE,), jnp.float32),
        pltpu.VMEM_SHARED((NPAD,), jnp.float32),
        pltpu.SemaphoreType.DMA,
    ],
    compiler_params=pltpu.CompilerParams(use_tc_tiling_on_sc=False),
)
def _sc_degree(dst_hbm, out0_hbm, out1_hbm, didx, ones, zbuf, acc, ssem):
  c = lax.axis_index("c")
  s = lax.axis_index("s")
  one16 = jnp.ones((16,), jnp.float32)
  z16 = jnp.zeros((16,), jnp.float32)

  def zb_body(i, carry):
    zbuf[pl.ds(i * 16, 16)] = z16
    return carry

  lax.fori_loop(0, ROWS_PER_TILE // 16, zb_body, 0)
  for i in range(LANES // 16):
    ones[pl.ds(i * 16, 16)] = one16

  pltpu.sync_copy(zbuf, acc.at[pl.ds(s * ROWS_PER_TILE, ROWS_PER_TILE)])
  plsc.subcore_barrier()

  base = (c * NS + s) * DEG_CHUNKS_PER_TILE

  def group(g, carry):
    gb = base + g * DEG_K
    pltpu.sync_copy(dst_hbm.at[pl.ds(gb, DEG_K)], didx)
    handles = []
    for b in range(DEG_K):
      handles.append(
          pltpu.async_copy(ones, acc.at[didx.at[b]], ssem, add=True))
    for h in handles:
      h.wait()
    return carry

  lax.fori_loop(0, DEG_GROUPS, group, 0)
  plsc.subcore_barrier()
  sl = pl.ds(s * ROWS_PER_TILE, ROWS_PER_TILE)

  @pl.when(c == 0)
  def _():
    pltpu.sync_copy(acc.at[sl], out0_hbm.at[sl])

  @pl.when(c == 1)
  def _():
    pltpu.sync_copy(acc.at[sl], out1_hbm.at[sl])


@functools.partial(
    pl.kernel,
    out_type=jax.ShapeDtypeStruct((NC, NPAD, 32), jnp.float32),
    mesh=_mesh,
    scratch_types=[
        pltpu.VMEM((AGG_BATCH, 2, LANES), jnp.int32),
        pltpu.VMEM((AGG_K, LANES, 32), jnp.float32),
        pltpu.VMEM_SHARED((NPAD, 32), jnp.float32),
        pltpu.SemaphoreType.DMA,
        pltpu.SemaphoreType.DMA,
    ],
    compiler_params=pltpu.CompilerParams(use_tc_tiling_on_sc=False),
)
def _sc_agg(table_hbm, eidx_hbm, out_hbm, ebuf, rows, acc, gsem, ssem):
  c = lax.axis_index("c")
  s = lax.axis_index("s")

  z16 = jnp.zeros((16,), jnp.float32)

  def zrow(i, carry):
    rows[0, i, 0:16] = z16
    rows[0, i, 16:32] = z16
    return carry

  lax.fori_loop(0, LANES, zrow, 0)

  def zcopy(k, carry):
    pltpu.sync_copy(
        rows.at[0], acc.at[pl.ds(s * ROWS_PER_TILE + k * LANES, LANES)])
    return carry

  lax.fori_loop(0, ROWS_PER_TILE // LANES, zcopy, 0)
  plsc.subcore_barrier()

  base = s * AGG_CHUNKS_PER_TILE

  def batch(i, carry):
    pltpu.sync_copy(eidx_hbm.at[c, pl.ds(base + i * AGG_BATCH, AGG_BATCH)],
                    ebuf)
    for r in range(AGG_BATCH // AGG_K):
      gh = []
      for b in range(AGG_K):
        gh.append(pltpu.async_copy(
            table_hbm.at[ebuf.at[r * AGG_K + b, 0]], rows.at[b], gsem))
      for h in gh:
        h.wait()
      sh = []
      for b in range(AGG_K):
        sh.append(pltpu.async_copy(
            rows.at[b], acc.at[ebuf.at[r * AGG_K + b, 1]], ssem, add=True))
      for h in sh:
        h.wait()
    return carry

  lax.fori_loop(0, AGG_BATCHES, batch, 0)
  plsc.subcore_barrier()
  pltpu.sync_copy(acc.at[pl.ds(s * ROWS_PER_TILE, ROWS_PER_TILE)],
                  out_hbm.at[c, pl.ds(s * ROWS_PER_TILE, ROWS_PER_TILE)])


_R = 2000


def _b1_body(x_ref, w_ref, d0_ref, d1_ref, g_ref, dinv_ref):
  deg = d0_ref[...] + d1_ref[...] + 1.0
  dinv = lax.rsqrt(deg)
  h = jnp.dot(x_ref[...], w_ref[...], preferred_element_type=jnp.float32)
  g = h * dinv
  g_ref[0] = g[:, :32]
  g_ref[1] = g[:, 32:]
  dinv_ref[...] = dinv


def _tc_b1(x, w_in, d0, d1):
  return pl.pallas_call(
      _b1_body,
      grid=(N // _R,),
      in_specs=[
          pl.BlockSpec((_R, D_IN), lambda i: (i, 0)),
          pl.BlockSpec((D_IN, D_H), lambda i: (0, 0)),
          pl.BlockSpec((_R, 1), lambda i: (i, 0)),
          pl.BlockSpec((_R, 1), lambda i: (i, 0)),
      ],
      out_specs=[
          pl.BlockSpec((NC, _R, 32), lambda i: (0, i, 0)),
          pl.BlockSpec((_R, 1), lambda i: (i, 0)),
      ],
      out_shape=[
          jax.ShapeDtypeStruct((NC, N, 32), jnp.float32),
          jax.ShapeDtypeStruct((N, 1), jnp.float32),
      ],
  )(x, w_in, d0, d1)


def _b2_body(agg_ref, g_ref, dinv_ref, b_ref, w_ref, o_ref):
  a = jnp.concatenate([agg_ref[0], agg_ref[1]], axis=1)
  g = jnp.concatenate([g_ref[0], g_ref[1]], axis=1)
  dinv = dinv_ref[...]
  h = jax.nn.relu(dinv * (a + g) + b_ref[...])
  g2 = jnp.dot(h, w_ref[...], preferred_element_type=jnp.float32)
  g2 = g2 * dinv
  o_ref[0] = g2[:, :32]
  o_ref[1] = g2[:, 32:]


def _tc_b2(agg, g, dinv, b, w):
  return pl.pallas_call(
      _b2_body,
      grid=(N // _R,),
      in_specs=[
          pl.BlockSpec((NC, _R, 32), lambda i: (0, i, 0)),
          pl.BlockSpec((NC, _R, 32), lambda i: (0, i, 0)),
          pl.BlockSpec((_R, 1), lambda i: (i, 0)),
          pl.BlockSpec((1, D_H), lambda i: (0, 0)),
          pl.BlockSpec((D_H, D_H), lambda i: (0, 0)),
      ],
      out_specs=pl.BlockSpec((NC, _R, 32), lambda i: (0, i, 0)),
      out_shape=jax.ShapeDtypeStruct((NC, N, 32), jnp.float32),
  )(agg, g, dinv, b, w)


def _b3_body(agg_ref, g_ref, dinv_ref, b_ref, o_ref):
  dinv = dinv_ref[...]
  for k in range(NC):
    h = jax.nn.relu(
        dinv * (agg_ref[k] + g_ref[k]) + b_ref[0, k * 32:(k + 1) * 32])
    o_ref[k] = h * dinv


def _tc_b3(agg, g, dinv, b):
  return pl.pallas_call(
      _b3_body,
      grid=(N // _R,),
      in_specs=[
          pl.BlockSpec((NC, _R, 32), lambda i: (0, i, 0)),
          pl.BlockSpec((NC, _R, 32), lambda i: (0, i, 0)),
          pl.BlockSpec((_R, 1), lambda i: (i, 0)),
          pl.BlockSpec((1, D_H), lambda i: (0, 0)),
      ],
      out_specs=pl.BlockSpec((NC, _R, 32), lambda i: (0, i, 0)),
      out_shape=jax.ShapeDtypeStruct((NC, N, 32), jnp.float32),
  )(agg, g, dinv, b)


def _b4_body(agg_ref, g_ref, dinv_ref, w_ref, b_ref, o_ref):
  a = jnp.concatenate([agg_ref[0], agg_ref[1]], axis=1)
  g = jnp.concatenate([g_ref[0], g_ref[1]], axis=1)
  dinv = dinv_ref[...]
  s3 = dinv * (a + g)
  o_ref[...] = jnp.dot(
      s3, w_ref[...], preferred_element_type=jnp.float32) + b_ref[...]


def _tc_b4(agg, g, dinv, w, b):
  return pl.pallas_call(
      _b4_body,
      grid=(N // _R,),
      in_specs=[
          pl.BlockSpec((NC, _R, 32), lambda i: (0, i, 0)),
          pl.BlockSpec((NC, _R, 32), lambda i: (0, i, 0)),
          pl.BlockSpec((_R, 1), lambda i: (i, 0)),
          pl.BlockSpec((D_H, D_OUT), lambda i: (0, 0)),
          pl.BlockSpec((1, D_OUT), lambda i: (0, 0)),
      ],
      out_specs=pl.BlockSpec((_R, D_OUT), lambda i: (i, 0)),
      out_shape=jax.ShapeDtypeStruct((N, D_OUT), jnp.float32),
  )(agg, g, dinv, w, b)


def kernel(x, edge_index, W_in, b_in, W_h, b_h, W_out, b_out):
  src = edge_index[0]
  dst = edge_index[1]
  pad = EPAD - E
  src_p = jnp.concatenate([src, jnp.zeros((pad,), jnp.int32)])
  dst_p = jnp.concatenate([dst, jnp.full((pad,), N, jnp.int32)])
  srcs = jnp.stack([src_p, src_p + N]).reshape(NC, CHUNKS, LANES)
  dst3 = dst_p.reshape(CHUNKS, LANES)
  eidx = jnp.stack(
      [srcs, jnp.broadcast_to(dst3, (NC, CHUNKS, LANES))], axis=2)

  d0, d1 = _sc_degree(dst3)
  g1, dinv = _tc_b1(x, W_in, d0.reshape(NPAD, 1), d1.reshape(NPAD, 1))

  agg1 = _sc_agg(g1.reshape(NC * N, 32), eidx)
  g2 = _tc_b2(agg1, g1, dinv, b_in.reshape(1, D_H), W_h)

  agg2 = _sc_agg(g2.reshape(NC * N, 32), eidx)
  g3 = _tc_b3(agg2, g2, dinv, b_h.reshape(1, D_H))

  agg3 = _sc_agg(g3.reshape(NC * N, 32), eidx)
  out = _tc_b4(agg3, g3, dinv, W_out, b_out.reshape(1, D_OUT))
  return out


# speedup vs baseline: 1.3128x; 1.0117x over previous
import functools

import jax
import jax.numpy as jnp
from jax import lax
from jax.experimental import pallas as pl
from jax.experimental.pallas import tpu as pltpu
from jax.experimental.pallas import tpu_sc as plsc

N = 50000
E = 800000
D_IN = 128
D_H = 64
D_OUT = 128

NC = 2
NS = 16
LANES = 128
CHUNKS = 6336
EPAD = CHUNKS * LANES
NPAD = 51200
ROWS_PER_TILE = NPAD // NS

AGG_CHUNKS_PER_TILE = CHUNKS // NS
AGG_BATCH = 36
AGG_BATCHES = AGG_CHUNKS_PER_TILE // AGG_BATCH
AGG_K = 2
AGG_GPB = AGG_BATCH // AGG_K

DEG_CHUNKS_PER_TILE = CHUNKS // (NC * NS)
DEG_K = 3
DEG_GROUPS = DEG_CHUNKS_PER_TILE // DEG_K

_mesh = plsc.VectorSubcoreMesh(
    core_axis_name="c", subcore_axis_name="s", num_cores=NC, num_subcores=NS)


@functools.partial(
    pl.kernel,
    out_type=(jax.ShapeDtypeStruct((NPAD,), jnp.float32),
              jax.ShapeDtypeStruct((NPAD,), jnp.float32)),
    mesh=_mesh,
    scratch_types=[
        pltpu.VMEM((DEG_K, LANES), jnp.int32),
        pltpu.VMEM((LANES,), jnp.float32),
        pltpu.VMEM((ROWS_PER_TILE,), jnp.float32),
        pltpu.VMEM_SHARED((NPAD,), jnp.float32),
        pltpu.SemaphoreType.DMA,
    ],
    compiler_params=pltpu.CompilerParams(use_tc_tiling_on_sc=False),
)
def _sc_degree(dst_hbm, out0_hbm, out1_hbm, didx, ones, zbuf, acc, ssem):
  c = lax.axis_index("c")
  s = lax.axis_index("s")
  one16 = jnp.ones((16,), jnp.float32)
  z16 = jnp.zeros((16,), jnp.float32)

  def zb_body(i, carry):
    zbuf[pl.ds(i * 16, 16)] = z16
    return carry

  lax.fori_loop(0, ROWS_PER_TILE // 16, zb_body, 0)
  for i in range(LANES // 16):
    ones[pl.ds(i * 16, 16)] = one16

  pltpu.sync_copy(zbuf, acc.at[pl.ds(s * ROWS_PER_TILE, ROWS_PER_TILE)])
  plsc.subcore_barrier()

  base = (c * NS + s) * DEG_CHUNKS_PER_TILE

  def group(g, carry):
    gb = base + g * DEG_K
    pltpu.sync_copy(dst_hbm.at[pl.ds(gb, DEG_K)], didx)
    handles = []
    for b in range(DEG_K):
      handles.append(
          pltpu.async_copy(ones, acc.at[didx.at[b]], ssem, add=True))
    for h in handles:
      h.wait()
    return carry

  lax.fori_loop(0, DEG_GROUPS, group, 0)
  plsc.subcore_barrier()
  sl = pl.ds(s * ROWS_PER_TILE, ROWS_PER_TILE)

  @pl.when(c == 0)
  def _():
    pltpu.sync_copy(acc.at[sl], out0_hbm.at[sl])

  @pl.when(c == 1)
  def _():
    pltpu.sync_copy(acc.at[sl], out1_hbm.at[sl])


@functools.partial(
    pl.kernel,
    out_type=jax.ShapeDtypeStruct((NC, NPAD, 32), jnp.float32),
    mesh=_mesh,
    scratch_types=[
        pltpu.VMEM((AGG_BATCH, 2, LANES), jnp.int32),
        pltpu.VMEM((AGG_K, LANES, 32), jnp.float32),
        pltpu.VMEM((AGG_K, LANES, 32), jnp.float32),
        pltpu.VMEM_SHARED((NPAD, 32), jnp.float32),
        pltpu.SemaphoreType.DMA,
        pltpu.SemaphoreType.DMA,
        pltpu.SemaphoreType.DMA,
        pltpu.SemaphoreType.DMA,
    ],
    compiler_params=pltpu.CompilerParams(use_tc_tiling_on_sc=False),
)
def _sc_agg(table_hbm, eidx_hbm, out_hbm, ebuf, rowsA, rowsB, acc,
            gsemA, gsemB, ssemA, ssemB):
  c = lax.axis_index("c")
  s = lax.axis_index("s")

  z16 = jnp.zeros((16,), jnp.float32)

  def zrow(i, carry):
    rowsA[0, i, 0:16] = z16
    rowsA[0, i, 16:32] = z16
    return carry

  lax.fori_loop(0, LANES, zrow, 0)

  def zcopy(k, carry):
    pltpu.sync_copy(
        rowsA.at[0], acc.at[pl.ds(s * ROWS_PER_TILE + k * LANES, LANES)])
    return carry

  lax.fori_loop(0, ROWS_PER_TILE // LANES, zcopy, 0)
  plsc.subcore_barrier()

  base = s * AGG_CHUNKS_PER_TILE

  banks = ((rowsA, gsemA, ssemA), (rowsB, gsemB, ssemB))

  def fire_gathers(j, rows, gsem):
    return [pltpu.async_copy(
        table_hbm.at[ebuf.at[j * AGG_K + b, 0]], rows.at[b], gsem)
            for b in range(AGG_K)]

  def fire_scatters(j, rows, ssem):
    return [pltpu.async_copy(
        rows.at[b], acc.at[ebuf.at[j * AGG_K + b, 1]], ssem, add=True)
            for b in range(AGG_K)]

  def batch(i, carry):
    pltpu.sync_copy(eidx_hbm.at[c, pl.ds(base + i * AGG_BATCH, AGG_BATCH)],
                    ebuf)
    gh = {0: fire_gathers(0, rowsA, gsemA)}
    sh = {}
    for j in range(AGG_GPB):
      rows, gsem, ssem = banks[j % 2]
      for h in gh.pop(j):
        h.wait()
      if j + 1 < AGG_GPB:
        nrows, ngsem, nssem = banks[(j + 1) % 2]
        if j - 1 in sh:
          for h in sh.pop(j - 1):
            h.wait()
        gh[j + 1] = fire_gathers(j + 1, nrows, ngsem)
      sh[j] = fire_scatters(j, rows, ssem)
    for hs in sh.values():
      for h in hs:
        h.wait()
    return carry

  lax.fori_loop(0, AGG_BATCHES, batch, 0)
  plsc.subcore_barrier()
  pltpu.sync_copy(acc.at[pl.ds(s * ROWS_PER_TILE, ROWS_PER_TILE)],
                  out_hbm.at[c, pl.ds(s * ROWS_PER_TILE, ROWS_PER_TILE)])


_R = 2000


def _b1_body(x_ref, w_ref, d0_ref, d1_ref, g_ref, dinv_ref):
  deg = d0_ref[...] + d1_ref[...] + 1.0
  dinv = lax.rsqrt(deg)
  h = jnp.dot(x_ref[...], w_ref[...], preferred_element_type=jnp.float32)
  g = h * dinv
  g_ref[0] = g[:, :32]
  g_ref[1] = g[:, 32:]
  dinv_ref[...] = dinv


def _tc_b1(x, w_in, d0, d1):
  return pl.pallas_call(
      _b1_body,
      grid=(N // _R,),
      in_specs=[
          pl.BlockSpec((_R, D_IN), lambda i: (i, 0)),
          pl.BlockSpec((D_IN, D_H), lambda i: (0, 0)),
          pl.BlockSpec((_R, 1), lambda i: (i, 0)),
          pl.BlockSpec((_R, 1), lambda i: (i, 0)),
      ],
      out_specs=[
          pl.BlockSpec((NC, _R, 32), lambda i: (0, i, 0)),
          pl.BlockSpec((_R, 1), lambda i: (i, 0)),
      ],
      out_shape=[
          jax.ShapeDtypeStruct((NC, N, 32), jnp.float32),
          jax.ShapeDtypeStruct((N, 1), jnp.float32),
      ],
  )(x, w_in, d0, d1)


def _b2_body(agg_ref, g_ref, dinv_ref, b_ref, w_ref, o_ref):
  a = jnp.concatenate([agg_ref[0], agg_ref[1]], axis=1)
  g = jnp.concatenate([g_ref[0], g_ref[1]], axis=1)
  dinv = dinv_ref[...]
  h = jax.nn.relu(dinv * (a + g) + b_ref[...])
  g2 = jnp.dot(h, w_ref[...], preferred_element_type=jnp.float32)
  g2 = g2 * dinv
  o_ref[0] = g2[:, :32]
  o_ref[1] = g2[:, 32:]


def _tc_b2(agg, g, dinv, b, w):
  return pl.pallas_call(
      _b2_body,
      grid=(N // _R,),
      in_specs=[
          pl.BlockSpec((NC, _R, 32), lambda i: (0, i, 0)),
          pl.BlockSpec((NC, _R, 32), lambda i: (0, i, 0)),
          pl.BlockSpec((_R, 1), lambda i: (i, 0)),
          pl.BlockSpec((1, D_H), lambda i: (0, 0)),
          pl.BlockSpec((D_H, D_H), lambda i: (0, 0)),
      ],
      out_specs=pl.BlockSpec((NC, _R, 32), lambda i: (0, i, 0)),
      out_shape=jax.ShapeDtypeStruct((NC, N, 32), jnp.float32),
  )(agg, g, dinv, b, w)


def _b3_body(agg_ref, g_ref, dinv_ref, b_ref, o_ref):
  dinv = dinv_ref[...]
  for k in range(NC):
    h = jax.nn.relu(
        dinv * (agg_ref[k] + g_ref[k]) + b_ref[0, k * 32:(k + 1) * 32])
    o_ref[k] = h * dinv


def _tc_b3(agg, g, dinv, b):
  return pl.pallas_call(
      _b3_body,
      grid=(N // _R,),
      in_specs=[
          pl.BlockSpec((NC, _R, 32), lambda i: (0, i, 0)),
          pl.BlockSpec((NC, _R, 32), lambda i: (0, i, 0)),
          pl.BlockSpec((_R, 1), lambda i: (i, 0)),
          pl.BlockSpec((1, D_H), lambda i: (0, 0)),
      ],
      out_specs=pl.BlockSpec((NC, _R, 32), lambda i: (0, i, 0)),
      out_shape=jax.ShapeDtypeStruct((NC, N, 32), jnp.float32),
  )(agg, g, dinv, b)


def _b4_body(agg_ref, g_ref, dinv_ref, w_ref, b_ref, o_ref):
  a = jnp.concatenate([agg_ref[0], agg_ref[1]], axis=1)
  g = jnp.concatenate([g_ref[0], g_ref[1]], axis=1)
  dinv = dinv_ref[...]
  s3 = dinv * (a + g)
  o_ref[...] = jnp.dot(
      s3, w_ref[...], preferred_element_type=jnp.float32) + b_ref[...]


def _tc_b4(agg, g, dinv, w, b):
  return pl.pallas_call(
      _b4_body,
      grid=(N // _R,),
      in_specs=[
          pl.BlockSpec((NC, _R, 32), lambda i: (0, i, 0)),
          pl.BlockSpec((NC, _R, 32), lambda i: (0, i, 0)),
          pl.BlockSpec((_R, 1), lambda i: (i, 0)),
          pl.BlockSpec((D_H, D_OUT), lambda i: (0, 0)),
          pl.BlockSpec((1, D_OUT), lambda i: (0, 0)),
      ],
      out_specs=pl.BlockSpec((_R, D_OUT), lambda i: (i, 0)),
      out_shape=jax.ShapeDtypeStruct((N, D_OUT), jnp.float32),
  )(agg, g, dinv, w, b)


def kernel(x, edge_index, W_in, b_in, W_h, b_h, W_out, b_out):
  src = edge_index[0]
  dst = edge_index[1]
  pad = EPAD - E
  src_p = jnp.concatenate([src, jnp.zeros((pad,), jnp.int32)])
  dst_p = jnp.concatenate([dst, jnp.full((pad,), N, jnp.int32)])
  srcs = jnp.stack([src_p, src_p + N]).reshape(NC, CHUNKS, LANES)
  dst3 = dst_p.reshape(CHUNKS, LANES)
  eidx = jnp.stack(
      [srcs, jnp.broadcast_to(dst3, (NC, CHUNKS, LANES))], axis=2)

  d0, d1 = _sc_degree(dst3)
  g1, dinv = _tc_b1(x, W_in, d0.reshape(NPAD, 1), d1.reshape(NPAD, 1))

  agg1 = _sc_agg(g1.reshape(NC * N, 32), eidx)
  g2 = _tc_b2(agg1, g1, dinv, b_in.reshape(1, D_H), W_h)

  agg2 = _sc_agg(g2.reshape(NC * N, 32), eidx)
  g3 = _tc_b3(agg2, g2, dinv, b_h.reshape(1, D_H))

  agg3 = _sc_agg(g3.reshape(NC * N, 32), eidx)
  out = _tc_b4(agg3, g3, dinv, W_out, b_out.reshape(1, D_OUT))
  return out


# speedup vs baseline: 1.3154x; 1.0020x over previous
import functools

import jax
import jax.numpy as jnp
from jax import lax
from jax.experimental import pallas as pl
from jax.experimental.pallas import tpu as pltpu
from jax.experimental.pallas import tpu_sc as plsc

N = 50000
E = 800000
D_IN = 128
D_H = 64
D_OUT = 128

NC = 2
NS = 16
LANES = 128
CHUNKS = 6336
EPAD = CHUNKS * LANES
NPAD = 51200
ROWS_PER_TILE = NPAD // NS

AGG_CHUNKS_PER_TILE = CHUNKS // NS
AGG_BATCH = 36
AGG_BATCHES = AGG_CHUNKS_PER_TILE // AGG_BATCH
AGG_K = 2
AGG_GPB = AGG_BATCH // AGG_K

DEG_CHUNKS_PER_TILE = CHUNKS // (NC * NS)
DEG_K = 3
DEG_GROUPS = DEG_CHUNKS_PER_TILE // DEG_K

_mesh = plsc.VectorSubcoreMesh(
    core_axis_name="c", subcore_axis_name="s", num_cores=NC, num_subcores=NS)


@functools.partial(
    pl.kernel,
    out_type=(jax.ShapeDtypeStruct((NPAD,), jnp.float32),
              jax.ShapeDtypeStruct((NPAD,), jnp.float32)),
    mesh=_mesh,
    scratch_types=[
        pltpu.VMEM((DEG_K, LANES), jnp.int32),
        pltpu.VMEM((LANES,), jnp.float32),
        pltpu.VMEM((ROWS_PER_TILE,), jnp.float32),
        pltpu.VMEM_SHARED((NPAD,), jnp.float32),
        pltpu.SemaphoreType.DMA,
    ],
    compiler_params=pltpu.CompilerParams(use_tc_tiling_on_sc=False),
)
def _sc_degree(dst_hbm, out0_hbm, out1_hbm, didx, ones, zbuf, acc, ssem):
  c = lax.axis_index("c")
  s = lax.axis_index("s")
  one16 = jnp.ones((16,), jnp.float32)
  z16 = jnp.zeros((16,), jnp.float32)

  def zb_body(i, carry):
    zbuf[pl.ds(i * 16, 16)] = z16
    return carry

  lax.fori_loop(0, ROWS_PER_TILE // 16, zb_body, 0)
  for i in range(LANES // 16):
    ones[pl.ds(i * 16, 16)] = one16

  pltpu.sync_copy(zbuf, acc.at[pl.ds(s * ROWS_PER_TILE, ROWS_PER_TILE)])
  plsc.subcore_barrier()

  base = (c * NS + s) * DEG_CHUNKS_PER_TILE

  def group(g, carry):
    gb = base + g * DEG_K
    pltpu.sync_copy(dst_hbm.at[pl.ds(gb, DEG_K)], didx)
    handles = []
    for b in range(DEG_K):
      handles.append(
          pltpu.async_copy(ones, acc.at[didx.at[b]], ssem, add=True))
    for h in handles:
      h.wait()
    return carry

  lax.fori_loop(0, DEG_GROUPS, group, 0)
  plsc.subcore_barrier()
  sl = pl.ds(s * ROWS_PER_TILE, ROWS_PER_TILE)

  @pl.when(c == 0)
  def _():
    pltpu.sync_copy(acc.at[sl], out0_hbm.at[sl])

  @pl.when(c == 1)
  def _():
    pltpu.sync_copy(acc.at[sl], out1_hbm.at[sl])


@functools.partial(
    pl.kernel,
    out_type=jax.ShapeDtypeStruct((NC, NPAD, 32), jnp.float32),
    mesh=_mesh,
    scratch_types=[
        pltpu.VMEM((AGG_BATCH, 2, LANES), jnp.int32),
        pltpu.VMEM((AGG_K, LANES, 32), jnp.float32),
        pltpu.VMEM((AGG_K, LANES, 32), jnp.float32),
        pltpu.VMEM_SHARED((NPAD, 32), jnp.float32),
        pltpu.SemaphoreType.DMA,
        pltpu.SemaphoreType.DMA,
        pltpu.SemaphoreType.DMA,
        pltpu.SemaphoreType.DMA,
    ],
    compiler_params=pltpu.CompilerParams(use_tc_tiling_on_sc=False),
)
def _sc_agg(table_hbm, eidx_hbm, out_hbm, ebuf, rowsA, rowsB, acc,
            gsemA, gsemB, ssemA, ssemB):
  c = lax.axis_index("c")
  s = lax.axis_index("s")

  z16 = jnp.zeros((16,), jnp.float32)

  def zrow(i, carry):
    rowsA[0, i, 0:16] = z16
    rowsA[0, i, 16:32] = z16
    return carry

  lax.fori_loop(0, LANES, zrow, 0)

  def zcopy(k, carry):
    pltpu.async_copy(
        rowsA.at[0], acc.at[pl.ds(s * ROWS_PER_TILE + k * LANES, LANES)],
        ssemA)
    return carry

  lax.fori_loop(0, ROWS_PER_TILE // LANES, zcopy, 0)

  def zdrain(k, carry):
    pltpu.make_async_copy(
        rowsA.at[0], acc.at[pl.ds(s * ROWS_PER_TILE + k * LANES, LANES)],
        ssemA).wait()
    return carry

  lax.fori_loop(0, ROWS_PER_TILE // LANES, zdrain, 0)
  plsc.subcore_barrier()

  base = s * AGG_CHUNKS_PER_TILE

  banks = ((rowsA, gsemA, ssemA), (rowsB, gsemB, ssemB))

  def fire_gathers(j, rows, gsem):
    return [pltpu.async_copy(
        table_hbm.at[ebuf.at[j * AGG_K + b, 0]], rows.at[b], gsem)
            for b in range(AGG_K)]

  def fire_scatters(j, rows, ssem):
    return [pltpu.async_copy(
        rows.at[b], acc.at[ebuf.at[j * AGG_K + b, 1]], ssem, add=True)
            for b in range(AGG_K)]

  def batch(i, carry):
    pltpu.sync_copy(eidx_hbm.at[c, pl.ds(base + i * AGG_BATCH, AGG_BATCH)],
                    ebuf)
    gh = {0: fire_gathers(0, rowsA, gsemA)}
    sh = {}
    for j in range(AGG_GPB):
      rows, gsem, ssem = banks[j % 2]
      for h in gh.pop(j):
        h.wait()
      if j + 1 < AGG_GPB:
        nrows, ngsem, nssem = banks[(j + 1) % 2]
        if j - 1 in sh:
          for h in sh.pop(j - 1):
            h.wait()
        gh[j + 1] = fire_gathers(j + 1, nrows, ngsem)
      sh[j] = fire_scatters(j, rows, ssem)
    for hs in sh.values():
      for h in hs:
        h.wait()
    return carry

  lax.fori_loop(0, AGG_BATCHES, batch, 0)
  plsc.subcore_barrier()
  pltpu.sync_copy(acc.at[pl.ds(s * ROWS_PER_TILE, ROWS_PER_TILE)],
                  out_hbm.at[c, pl.ds(s * ROWS_PER_TILE, ROWS_PER_TILE)])


_R = 2000


def _b1_body(x_ref, w_ref, d0_ref, d1_ref, g_ref, dinv_ref):
  deg = d0_ref[...] + d1_ref[...] + 1.0
  dinv = lax.rsqrt(deg)
  h = jnp.dot(x_ref[...], w_ref[...], preferred_element_type=jnp.float32)
  g = h * dinv
  g_ref[0] = g[:, :32]
  g_ref[1] = g[:, 32:]
  dinv_ref[...] = dinv


def _tc_b1(x, w_in, d0, d1):
  return pl.pallas_call(
      _b1_body,
      grid=(N // _R,),
      in_specs=[
          pl.BlockSpec((_R, D_IN), lambda i: (i, 0)),
          pl.BlockSpec((D_IN, D_H), lambda i: (0, 0)),
          pl.BlockSpec((_R, 1), lambda i: (i, 0)),
          pl.BlockSpec((_R, 1), lambda i: (i, 0)),
      ],
      out_specs=[
          pl.BlockSpec((NC, _R, 32), lambda i: (0, i, 0)),
          pl.BlockSpec((_R, 1), lambda i: (i, 0)),
      ],
      out_shape=[
          jax.ShapeDtypeStruct((NC, N, 32), jnp.float32),
          jax.ShapeDtypeStruct((N, 1), jnp.float32),
      ],
  )(x, w_in, d0, d1)


def _b2_body(agg_ref, g_ref, dinv_ref, b_ref, w_ref, o_ref):
  a = jnp.concatenate([agg_ref[0], agg_ref[1]], axis=1)
  g = jnp.concatenate([g_ref[0], g_ref[1]], axis=1)
  dinv = dinv_ref[...]
  h = jax.nn.relu(dinv * (a + g) + b_ref[...])
  g2 = jnp.dot(h, w_ref[...], preferred_element_type=jnp.float32)
  g2 = g2 * dinv
  o_ref[0] = g2[:, :32]
  o_ref[1] = g2[:, 32:]


def _tc_b2(agg, g, dinv, b, w):
  return pl.pallas_call(
      _b2_body,
      grid=(N // _R,),
      in_specs=[
          pl.BlockSpec((NC, _R, 32), lambda i: (0, i, 0)),
          pl.BlockSpec((NC, _R, 32), lambda i: (0, i, 0)),
          pl.BlockSpec((_R, 1), lambda i: (i, 0)),
          pl.BlockSpec((1, D_H), lambda i: (0, 0)),
          pl.BlockSpec((D_H, D_H), lambda i: (0, 0)),
      ],
      out_specs=pl.BlockSpec((NC, _R, 32), lambda i: (0, i, 0)),
      out_shape=jax.ShapeDtypeStruct((NC, N, 32), jnp.float32),
  )(agg, g, dinv, b, w)


def _b3_body(agg_ref, g_ref, dinv_ref, b_ref, o_ref):
  dinv = dinv_ref[...]
  for k in range(NC):
    h = jax.nn.relu(
        dinv * (agg_ref[k] + g_ref[k]) + b_ref[0, k * 32:(k + 1) * 32])
    o_ref[k] = h * dinv


def _tc_b3(agg, g, dinv, b):
  return pl.pallas_call(
      _b3_body,
      grid=(N // _R,),
      in_specs=[
          pl.BlockSpec((NC, _R, 32), lambda i: (0, i, 0)),
          pl.BlockSpec((NC, _R, 32), lambda i: (0, i, 0)),
          pl.BlockSpec((_R, 1), lambda i: (i, 0)),
          pl.BlockSpec((1, D_H), lambda i: (0, 0)),
      ],
      out_specs=pl.BlockSpec((NC, _R, 32), lambda i: (0, i, 0)),
      out_shape=jax.ShapeDtypeStruct((NC, N, 32), jnp.float32),
  )(agg, g, dinv, b)


def _b4_body(agg_ref, g_ref, dinv_ref, w_ref, b_ref, o_ref):
  a = jnp.concatenate([agg_ref[0], agg_ref[1]], axis=1)
  g = jnp.concatenate([g_ref[0], g_ref[1]], axis=1)
  dinv = dinv_ref[...]
  s3 = dinv * (a + g)
  o_ref[...] = jnp.dot(
      s3, w_ref[...], preferred_element_type=jnp.float32) + b_ref[...]


def _tc_b4(agg, g, dinv, w, b):
  return pl.pallas_call(
      _b4_body,
      grid=(N // _R,),
      in_specs=[
          pl.BlockSpec((NC, _R, 32), lambda i: (0, i, 0)),
          pl.BlockSpec((NC, _R, 32), lambda i: (0, i, 0)),
          pl.BlockSpec((_R, 1), lambda i: (i, 0)),
          pl.BlockSpec((D_H, D_OUT), lambda i: (0, 0)),
          pl.BlockSpec((1, D_OUT), lambda i: (0, 0)),
      ],
      out_specs=pl.BlockSpec((_R, D_OUT), lambda i: (i, 0)),
      out_shape=jax.ShapeDtypeStruct((N, D_OUT), jnp.float32),
  )(agg, g, dinv, w, b)


def kernel(x, edge_index, W_in, b_in, W_h, b_h, W_out, b_out):
  src = edge_index[0]
  dst = edge_index[1]
  pad = EPAD - E
  src_p = jnp.concatenate([src, jnp.zeros((pad,), jnp.int32)])
  dst_p = jnp.concatenate([dst, jnp.full((pad,), N, jnp.int32)])
  srcs = jnp.stack([src_p, src_p + N]).reshape(NC, CHUNKS, LANES)
  dst3 = dst_p.reshape(CHUNKS, LANES)
  eidx = jnp.stack(
      [srcs, jnp.broadcast_to(dst3, (NC, CHUNKS, LANES))], axis=2)

  d0, d1 = _sc_degree(dst3)
  g1, dinv = _tc_b1(x, W_in, d0.reshape(NPAD, 1), d1.reshape(NPAD, 1))

  agg1 = _sc_agg(g1.reshape(NC * N, 32), eidx)
  g2 = _tc_b2(agg1, g1, dinv, b_in.reshape(1, D_H), W_h)

  agg2 = _sc_agg(g2.reshape(NC * N, 32), eidx)
  g3 = _tc_b3(agg2, g2, dinv, b_h.reshape(1, D_H))

  agg3 = _sc_agg(g3.reshape(NC * N, 32), eidx)
  out = _tc_b4(agg3, g3, dinv, W_out, b_out.reshape(1, D_OUT))
  return out
